# baseline (speedup 1.0000x reference)
import math
import functools
import jax
import jax.numpy as jnp
from jax import lax
from jax.experimental import pallas as pl
from jax.experimental.pallas import tpu as pltpu
from jax.experimental.pallas import tpu_sc as plsc

B = 2
N = 400
F = 4
CS = 8
EPG = N * N
E = B * EPG
NN = B * N

NP = 1024
NW = 32
EPT = E // NW
NG = NP // 16
NGV = NN // 16
COLS = NP // 16
RC = 16
def _seg_plan(in_ch):
    segs = [2000] * 5 if in_ch == F else [1008] * 9 + [928]
    offs = [sum(segs[:k]) for k in range(len(segs))]
    unroll = 2 if in_ch == F else 1
    return segs, offs, unroll


def _sc_layer_body(in_ch, out_ch, ipp, want_counts, want_h, *refs):
    finalize = ipp > 0
    it = iter(refs)
    if finalize:
        pprev_r = next(it); cntp_r = next(it); hroot_r = next(it)
        rs_r = next(it); bi_r = next(it)
    else:
        xT_r = next(it)
    zT_r = next(it); src_r = next(it); dst_r = next(it)
    p_out = next(it)
    cnt_out = next(it) if want_counts else None
    h_out = next(it) if want_h else None
    hbuf = next(it); srcb = next(it); dstb = next(it)
    wbuf = next(it); sems = next(it)
    accl = next(it); acc2d = next(it)
    if finalize:
        pv = next(it); cv = next(it); hv = next(it)
        rsv = next(it); biv = next(it)
    if want_counts:
        cnt1d = next(it)
        acclc = next(it)
    SA = next(it)
    SA2 = next(it) if want_counts else None
    rbuf = next(it); rsum = next(it)
    if want_counts:
        rbufc = next(it); rsumc = next(it)

    cid = lax.axis_index("c")
    sid = lax.axis_index("s")
    wid = sid * 2 + cid
    e0 = wid * EPT

    pltpu.sync_copy(src_r.at[pl.ds(e0, EPT)], srcb)
    pltpu.sync_copy(dst_r.at[pl.ds(e0, EPT)], dstb)
    if finalize:
        pltpu.sync_copy(pprev_r, pv)
        pltpu.sync_copy(cntp_r, cv)
        pltpu.sync_copy(hroot_r, hv)
        pltpu.sync_copy(rs_r, rsv)
        pltpu.sync_copy(bi_r, biv)
    else:
        pltpu.sync_copy(xT_r, hbuf)

    if finalize:
        rvecs = [[rsv[c, i, :] for i in range(ipp)] for c in range(in_ch)]
        bvecs = [biv[c, :] for c in range(in_ch)]

        @pl.loop(0, NG)
        def _fin(g):
            sl = pl.ds(g * 16, 16)
            cnt = cv[0, sl] + cv[1, sl]
            rc = 1.0 / jnp.maximum(cnt, 1.0)
            for c in range(in_ch):
                s = (pv[0, c, sl] + pv[1, c, sl]) * rc + bvecs[c]
                for i in range(ipp):
                    s = s + hv[i, sl] * rvecs[c][i]
                hbuf[c, sl] = s

    z16 = jnp.zeros((16,), jnp.float32)

    @pl.loop(0, NG)
    def _zero2d(g):
        sl = pl.ds(g * 16, 16)
        for o in range(out_ch):
            acc2d[o, sl] = z16
        if want_counts:
            cnt1d[sl] = z16

    lane = lax.iota(jnp.int32, 16)
    ones = jnp.full((16,), 1.0, jnp.float32)

    SEGS, SOFF, UNR = _seg_plan(in_ch)
    jidx = [jnp.full((16,), j, jnp.int32) for j in range(2)]
    for p in range(out_ch // 2):
        o0 = 2 * p
        fuse_counts = want_counts and p == 0

        @pl.loop(0, NGV, unroll=2)
        def _zl(g):
            sl = pl.ds(g * 16, 16)
            for l in range(16):
                accl[0, l, sl] = z16
                accl[1, l, sl] = z16
                if fuse_counts:
                    acclc[l, sl] = z16

        copies = [None] * len(SEGS)

        def _start(s):
            return pltpu.async_copy(
                zT_r.at[pl.ds(o0 * in_ch, 2 * in_ch),
                        pl.ds(e0 + SOFF[s], SEGS[s])],
                wbuf.at[s % 2, :, pl.ds(0, SEGS[s])], sems[s % 2])

        copies[0] = _start(0)
        for s in range(len(SEGS)):
            copies[s].wait()
            if s + 1 < len(SEGS):
                copies[s + 1] = _start(s + 1)
            bsl = s % 2

            @pl.loop(0, SEGS[s] // 16, unroll=UNR)
            def _edges(t):
                sle = pl.ds(SOFF[s] + t * 16, 16)
                slw = pl.ds(t * 16, 16)
                sv = srcb[sle]
                dv = dstb[sle]
                msg0 = z16
                msg1 = z16
                for i in range(in_ch):
                    xj = plsc.load_gather(
                        hbuf, [jnp.full((16,), i, jnp.int32), sv])
                    msg0 = msg0 + xj * wbuf[bsl, i, slw]
                    msg1 = msg1 + xj * wbuf[bsl, in_ch + i, slw]
                plsc.addupdate_scatter(accl, [jidx[0], lane, dv], msg0)
                plsc.addupdate_scatter(accl, [jidx[1], lane, dv], msg1)
                if fuse_counts:
                    plsc.addupdate_scatter(acclc, [lane, dv], ones)

        @pl.loop(0, NGV, unroll=2)
        def _red(g):
            sl = pl.ds(g * 16, 16)
            for j in range(2):
                s = accl[j, 0, sl]
                for l in range(1, 16):
                    s = s + accl[j, l, sl]
                acc2d[o0 + j, sl] = s
            if fuse_counts:
                c = acclc[0, sl]
                for l in range(1, 16):
                    c = c + acclc[l, sl]
                cnt1d[sl] = c

    pltpu.sync_copy(acc2d, SA.at[sid])
    if want_counts:
        pltpu.sync_copy(cnt1d, SA2.at[sid])
    plsc.subcore_barrier()

    for q in range(COLS // RC):
        c0 = sid * COLS + q * RC
        pltpu.sync_copy(SA.at[:, :, pl.ds(c0, RC)], rbuf)
        for o in range(out_ch):
            s = rbuf[0, o, :]
            for t in range(1, 16):
                s = s + rbuf[t, o, :]
            rsum[o, :] = s
        pltpu.sync_copy(rsum, p_out.at[cid, :, pl.ds(c0, RC)])
        if want_counts:
            pltpu.sync_copy(SA2.at[:, pl.ds(c0, RC)], rbufc)
            s = rbufc[0, :]
            for t in range(1, 16):
                s = s + rbufc[t, :]
            rsumc[:] = s
            pltpu.sync_copy(rsumc, cnt_out.at[cid, pl.ds(c0, RC)])

    if want_h:
        @pl.when(jnp.logical_and(cid == 0, sid == 0))
        def _wh():
            pltpu.sync_copy(hbuf, h_out)


def _sc_layer(in_ch, out_ch, ipp, want_counts, want_h):
    finalize = ipp > 0
    f32 = jnp.float32
    out_type = [jax.ShapeDtypeStruct((2, out_ch, NP), f32)]
    if want_counts:
        out_type.append(jax.ShapeDtypeStruct((2, NP), f32))
    if want_h:
        out_type.append(jax.ShapeDtypeStruct((in_ch, NP), f32))
    scratch = [
        pltpu.VMEM((in_ch, NP), f32),
        pltpu.VMEM((EPT,), jnp.int32),
        pltpu.VMEM((EPT,), jnp.int32),
        pltpu.VMEM((2, 2 * in_ch, _seg_plan(in_ch)[0][0]), f32),
        (pltpu.SemaphoreType.DMA, pltpu.SemaphoreType.DMA),
        pltpu.VMEM((2, 16, NN), f32),
        pltpu.VMEM((out_ch, NP), f32),
    ]
    if finalize:
        scratch += [
            pltpu.VMEM((2, in_ch, NP), f32),
            pltpu.VMEM((2, NP), f32),
            pltpu.VMEM((ipp, NP), f32),
            pltpu.VMEM((in_ch, ipp, 16), f32),
            pltpu.VMEM((in_ch, 16), f32),
        ]
    if want_counts:
        scratch.append(pltpu.VMEM((NP,), f32))
        scratch.append(pltpu.VMEM((16, NN), f32))
    scratch.append(pltpu.VMEM_SHARED((16, out_ch, NP), f32))
    if want_counts:
        scratch.append(pltpu.VMEM_SHARED((16, NP), f32))
    scratch += [
        pltpu.VMEM((16, out_ch, RC), f32),
        pltpu.VMEM((out_ch, RC), f32),
    ]
    if want_counts:
        scratch += [
            pltpu.VMEM((16, RC), f32),
            pltpu.VMEM((RC,), f32),
        ]
    mesh = plsc.VectorSubcoreMesh(core_axis_name="c", subcore_axis_name="s")
    body = functools.partial(_sc_layer_body, in_ch, out_ch, ipp,
                             want_counts, want_h)
    return pl.kernel(body, out_type=out_type, mesh=mesh,
                     scratch_types=scratch,
                     compiler_params=pltpu.CompilerParams(
                         use_tc_tiling_on_sc=False,
                         needs_layout_passes=False))


def _rsplat(root, bias, ipp, c):
    rs = jnp.broadcast_to(root.T[..., None], (c, ipp, 16))
    bi = jnp.broadcast_to(bias[:, None], (c, 16))
    return rs, bi


_EC = 12800


def _zmlp3_kernel(ea_ref, Wt_ref, b_ref, z1_ref, z2_ref, z3_ref):
    ea = ea_ref[...]
    z = jax.lax.dot_general(
        Wt_ref[...], ea, (((1,), (1,)), ((), ())),
        preferred_element_type=jnp.float32) + b_ref[...]
    z = jnp.maximum(z, 0.0)
    z1_ref[...] = z[:32]
    z2_ref[...] = z[32:96]
    z3_ref[...] = z[96:]


def _zmlp3(ea, nn1_W, nn1_b, nn2_W, nn2_b, nn3_W, nn3_b):
    def prep(W, b, in_ch, out_ch):
        K = out_ch * in_ch
        Wp = W.reshape(F, in_ch, out_ch).transpose(2, 1, 0)
        return Wp.reshape(K, F), b.reshape(in_ch, out_ch).T.reshape(K, 1)

    W1p, b1p = prep(nn1_W, nn1_b, F, CS)
    W2p, b2p = prep(nn2_W, nn2_b, CS, CS)
    W3p, b3p = prep(nn3_W, nn3_b, CS, F)
    Wt = jnp.concatenate([W1p, W2p, W3p], axis=0)
    bt = jnp.concatenate([b1p, b2p, b3p], axis=0)
    grid = (E // _EC,)
    return pl.pallas_call(
        _zmlp3_kernel,
        grid=grid,
        in_specs=[
            pl.BlockSpec((_EC, F), lambda i: (i, 0)),
            pl.BlockSpec((128, F), lambda i: (0, 0)),
            pl.BlockSpec((128, 1), lambda i: (0, 0)),
        ],
        out_specs=[
            pl.BlockSpec((32, _EC), lambda i: (0, i)),
            pl.BlockSpec((64, _EC), lambda i: (0, i)),
            pl.BlockSpec((32, _EC), lambda i: (0, i)),
        ],
        out_shape=[
            jax.ShapeDtypeStruct((32, E), jnp.float32),
            jax.ShapeDtypeStruct((64, E), jnp.float32),
            jax.ShapeDtypeStruct((32, E), jnp.float32),
        ],
    )(ea, Wt, bt)


def _tail_kernel(p3_ref, cntp_ref, h2T_ref, root3t_ref, bias3_ref,
                 ts_ref, noisyT_ref,
                 tl1_W_ref, tl1_b_ref, tl2_W_ref, tl2_b_ref,
                 fm1_Wt_ref, fm1_b_ref, fm2_Wt_ref, fm2_b_ref,
                 fm3_Wt_ref, fm3_b_ref,
                 bn_gamma_ref, bn_beta_ref, out_ref):
    cnt = (cntp_ref[0:1, :NN] + cntp_ref[1:2, :NN])
    rc = 1.0 / jnp.maximum(cnt, 1.0)
    h2T = h2T_ref[...][:, :NN]
    mean = (p3_ref[0, :, :NN] + p3_ref[1, :, :NN]) * rc
    h3T = mean + jnp.dot(root3t_ref[...], h2T,
                         preferred_element_type=jnp.float32) + bias3_ref[...]

    t = ts_ref[...].astype(jnp.float32)
    half = N // 2
    k = lax.broadcasted_iota(jnp.int32, (1, half), 1).astype(jnp.float32)
    freqs = jnp.exp(k * (-(math.log(10000.0) / (half - 1))))
    ang = t * freqs
    se = jnp.concatenate([jnp.sin(ang), jnp.cos(ang)], axis=-1)
    pre = jnp.dot(se, tl1_W_ref[...], preferred_element_type=jnp.float32) + tl1_b_ref[...]
    g = 0.5 * pre * (1.0 + lax.erf(pre / jnp.sqrt(2.0).astype(jnp.float32)))
    te = jnp.dot(g, tl2_W_ref[...], preferred_element_type=jnp.float32) + tl2_b_ref[...]

    m1 = jax.nn.relu(jnp.dot(fm1_Wt_ref[...], h3T, preferred_element_type=jnp.float32)
                     + fm1_b_ref[...])
    m2 = jax.nn.relu(jnp.dot(fm2_Wt_ref[...], m1, preferred_element_type=jnp.float32)
                     + fm2_b_ref[...])
    m3 = jax.nn.sigmoid(jnp.dot(fm3_Wt_ref[...], m2, preferred_element_type=jnp.float32)
                        + fm3_b_ref[...])
    mapped = m3.reshape(F, B, N) * te[None, :, :]

    noisyT = noisyT_ref[...]
    mu = jnp.sum(noisyT, axis=(0, 1), keepdims=True) / (F * B)
    var = jnp.sum((noisyT - mu) ** 2, axis=(0, 1), keepdims=True) / (F * B)
    bn = (noisyT - mu) * lax.rsqrt(var + 1e-5)
    bn = bn * bn_gamma_ref[...][None, None, :] + bn_beta_ref[...][None, None, :]
    out_ref[...] = bn - mapped


def kernel(noisy_x, source_x, edge_index, source_edge_attr, timesteps,
           nn1_W, nn1_b, nn2_W, nn2_b, nn3_W, nn3_b,
           root1, bias1, root2, bias2, root3, bias3,
           tl1_W, tl1_b, tl2_W, tl2_b,
           fm1_W, fm1_b, fm2_W, fm2_b, fm3_W, fm3_b,
           bn_gamma, bn_beta):
    f32 = jnp.float32
    x = source_x.reshape(NN, F)
    xTp = jnp.zeros((F, NP), f32).at[:, :NN].set(x.T)
    src = edge_index[0]
    dst = edge_index[1]

    zT1, zT2, zT3 = _zmlp3(source_edge_attr.reshape(E, F),
                           nn1_W, nn1_b, nn2_W, nn2_b, nn3_W, nn3_b)
    rs1, bi1 = _rsplat(root1, bias1, F, CS)
    rs2, bi2 = _rsplat(root2, bias2, CS, CS)

    p1, cntp = _sc_layer(F, CS, 0, True, False)(
        xTp, zT1, src, dst)
    p2, h1T = _sc_layer(CS, CS, F, False, True)(
        p1, cntp, xTp, rs1, bi1, zT2, src, dst)
    p3, h2T = _sc_layer(CS, F, CS, False, True)(
        p2, cntp, h1T, rs2, bi2, zT3, src, dst)

    noisyT = noisy_x.transpose(2, 0, 1)
    outT = pl.pallas_call(
        _tail_kernel,
        out_shape=jax.ShapeDtypeStruct((F, B, N), f32),
    )(p3, cntp, h2T, root3.T, bias3.reshape(F, 1),
      timesteps.reshape(B, 1), noisyT,
      tl1_W, tl1_b.reshape(1, N), tl2_W, tl2_b.reshape(1, N),
      fm1_W.T, fm1_b.reshape(128, 1), fm2_W.T, fm2_b.reshape(128, 1),
      fm3_W.T, fm3_b.reshape(F, 1), bn_gamma, bn_beta)
    return outT.transpose(1, 2, 0)


# speedup vs baseline: 1.2941x; 1.2941x over previous
import math
import functools
import jax
import jax.numpy as jnp
from jax import lax
from jax.experimental import pallas as pl
from jax.experimental.pallas import tpu as pltpu
from jax.experimental.pallas import tpu_sc as plsc

B = 2
N = 400
F = 4
CS = 8
EPG = N * N
E = B * EPG
NN = B * N

NP = 1024
NW = 32
EPT = E // NW
NG = NP // 16
NGV = NN // 16
COLS = NP // 16
RC = 16
def _seg_plan(in_ch):
    segs = [2000] * 5 if in_ch == F else [1008] * 9 + [928]
    offs = [sum(segs[:k]) for k in range(len(segs))]
    unroll = 2 if in_ch == F else 1
    return segs, offs, unroll


def _sc_layer_body(in_ch, out_ch, ipp, want_counts, want_h, *refs):
    finalize = ipp > 0
    it = iter(refs)
    if finalize:
        pprev_r = next(it); cntp_r = next(it); hroot_r = next(it)
        rs_r = next(it); bi_r = next(it)
    else:
        xT_r = next(it)
    zT_r = next(it); src_r = next(it); dst_r = next(it)
    p_out = next(it)
    cnt_out = next(it) if want_counts else None
    h_out = next(it) if want_h else None
    hbuf = next(it); srcb = next(it); dstb = next(it)
    wbuf = next(it); sems = next(it)
    accl = next(it); acc2d = next(it)
    if finalize:
        pv = next(it); cv = next(it); hv = next(it)
        rsv = next(it); biv = next(it)
    if want_counts:
        cnt1d = next(it)
        acclc = next(it)
    SA = next(it)
    SA2 = next(it) if want_counts else None
    rbuf = next(it); rsum = next(it)
    if want_counts:
        rbufc = next(it); rsumc = next(it)

    cid = lax.axis_index("c")
    sid = lax.axis_index("s")
    wid = sid * 2 + cid
    e0 = wid * EPT

    pltpu.sync_copy(src_r.at[pl.ds(e0, EPT)], srcb)
    pltpu.sync_copy(dst_r.at[pl.ds(e0, EPT)], dstb)
    if finalize:
        pltpu.sync_copy(pprev_r, pv)
        pltpu.sync_copy(cntp_r, cv)
        pltpu.sync_copy(hroot_r, hv)
        pltpu.sync_copy(rs_r, rsv)
        pltpu.sync_copy(bi_r, biv)
    else:
        pltpu.sync_copy(xT_r, hbuf)

    if finalize:
        rvecs = [[rsv[c, i, :] for i in range(ipp)] for c in range(in_ch)]
        bvecs = [biv[c, :] for c in range(in_ch)]

        @pl.loop(0, NG)
        def _fin(g):
            sl = pl.ds(g * 16, 16)
            cnt = cv[0, sl] + cv[1, sl]
            rc = 1.0 / jnp.maximum(cnt, 1.0)
            for c in range(in_ch):
                s = (pv[0, c, sl] + pv[1, c, sl]) * rc + bvecs[c]
                for i in range(ipp):
                    s = s + hv[i, sl] * rvecs[c][i]
                hbuf[c, sl] = s

    z16 = jnp.zeros((16,), jnp.float32)

    @pl.loop(0, NG)
    def _zero2d(g):
        sl = pl.ds(g * 16, 16)
        for o in range(out_ch):
            acc2d[o, sl] = z16
        if want_counts:
            cnt1d[sl] = z16

    lane = lax.iota(jnp.int32, 16)
    ones = jnp.full((16,), 1.0, jnp.float32)

    SEGS, SOFF, UNR = _seg_plan(in_ch)
    jidx = [jnp.full((16,), j, jnp.int32) for j in range(2)]
    for p in range(out_ch // 2):
        o0 = 2 * p
        fuse_counts = want_counts and p == 0

        @pl.loop(0, NGV, unroll=2)
        def _zl(g):
            sl = pl.ds(g * 16, 16)
            for l in range(16):
                accl[0, l, sl] = z16
                accl[1, l, sl] = z16
                if fuse_counts:
                    acclc[l, sl] = z16

        copies = [None] * len(SEGS)

        def _start(s):
            return pltpu.async_copy(
                zT_r.at[pl.ds(o0 * in_ch, 2 * in_ch),
                        pl.ds(e0 + SOFF[s], SEGS[s])],
                wbuf.at[s % 2, :, pl.ds(0, SEGS[s])], sems[s % 2])

        copies[0] = _start(0)
        for s in range(len(SEGS)):
            copies[s].wait()
            if s + 1 < len(SEGS):
                copies[s + 1] = _start(s + 1)
            bsl = s % 2

            @pl.loop(0, SEGS[s] // 16, unroll=UNR)
            def _edges(t):
                sle = pl.ds(SOFF[s] + t * 16, 16)
                slw = pl.ds(t * 16, 16)
                sv = srcb[sle]
                dv = dstb[sle]
                msg0 = z16
                msg1 = z16
                for i in range(in_ch):
                    xj = plsc.load_gather(
                        hbuf, [jnp.full((16,), i, jnp.int32), sv])
                    msg0 = msg0 + xj * wbuf[bsl, i, slw]
                    msg1 = msg1 + xj * wbuf[bsl, in_ch + i, slw]
                plsc.addupdate_scatter(accl, [jidx[0], lane, dv], msg0)
                plsc.addupdate_scatter(accl, [jidx[1], lane, dv], msg1)
                if fuse_counts:
                    plsc.addupdate_scatter(acclc, [lane, dv], ones)

        @pl.loop(0, NGV, unroll=2)
        def _red(g):
            sl = pl.ds(g * 16, 16)
            for j in range(2):
                s = accl[j, 0, sl]
                for l in range(1, 16):
                    s = s + accl[j, l, sl]
                acc2d[o0 + j, sl] = s
            if fuse_counts:
                c = acclc[0, sl]
                for l in range(1, 16):
                    c = c + acclc[l, sl]
                cnt1d[sl] = c

    pltpu.sync_copy(acc2d, SA.at[sid])
    if want_counts:
        pltpu.sync_copy(cnt1d, SA2.at[sid])
    plsc.subcore_barrier()

    for q in range(COLS // RC):
        c0 = sid * COLS + q * RC
        pltpu.sync_copy(SA.at[:, :, pl.ds(c0, RC)], rbuf)
        for o in range(out_ch):
            s = rbuf[0, o, :]
            for t in range(1, 16):
                s = s + rbuf[t, o, :]
            rsum[o, :] = s
        pltpu.sync_copy(rsum, p_out.at[cid, :, pl.ds(c0, RC)])
        if want_counts:
            pltpu.sync_copy(SA2.at[:, pl.ds(c0, RC)], rbufc)
            s = rbufc[0, :]
            for t in range(1, 16):
                s = s + rbufc[t, :]
            rsumc[:] = s
            pltpu.sync_copy(rsumc, cnt_out.at[cid, pl.ds(c0, RC)])

    if want_h:
        @pl.when(jnp.logical_and(cid == 0, sid == 0))
        def _wh():
            pltpu.sync_copy(hbuf, h_out)


def _sc_layer(in_ch, out_ch, ipp, want_counts, want_h):
    finalize = ipp > 0
    f32 = jnp.float32
    out_type = [jax.ShapeDtypeStruct((2, out_ch, NP), f32)]
    if want_counts:
        out_type.append(jax.ShapeDtypeStruct((2, NP), f32))
    if want_h:
        out_type.append(jax.ShapeDtypeStruct((in_ch, NP), f32))
    scratch = [
        pltpu.VMEM((in_ch, NP), f32),
        pltpu.VMEM((EPT,), jnp.int32),
        pltpu.VMEM((EPT,), jnp.int32),
        pltpu.VMEM((2, 2 * in_ch, _seg_plan(in_ch)[0][0]), f32),
        (pltpu.SemaphoreType.DMA, pltpu.SemaphoreType.DMA),
        pltpu.VMEM((2, 16, NN), f32),
        pltpu.VMEM((out_ch, NP), f32),
    ]
    if finalize:
        scratch += [
            pltpu.VMEM((2, in_ch, NP), f32),
            pltpu.VMEM((2, NP), f32),
            pltpu.VMEM((ipp, NP), f32),
            pltpu.VMEM((in_ch, ipp, 16), f32),
            pltpu.VMEM((in_ch, 16), f32),
        ]
    if want_counts:
        scratch.append(pltpu.VMEM((NP,), f32))
        scratch.append(pltpu.VMEM((16, NN), f32))
    scratch.append(pltpu.VMEM_SHARED((16, out_ch, NP), f32))
    if want_counts:
        scratch.append(pltpu.VMEM_SHARED((16, NP), f32))
    scratch += [
        pltpu.VMEM((16, out_ch, RC), f32),
        pltpu.VMEM((out_ch, RC), f32),
    ]
    if want_counts:
        scratch += [
            pltpu.VMEM((16, RC), f32),
            pltpu.VMEM((RC,), f32),
        ]
    mesh = plsc.VectorSubcoreMesh(core_axis_name="c", subcore_axis_name="s")
    body = functools.partial(_sc_layer_body, in_ch, out_ch, ipp,
                             want_counts, want_h)
    return pl.kernel(body, out_type=out_type, mesh=mesh,
                     scratch_types=scratch,
                     compiler_params=pltpu.CompilerParams(
                         use_tc_tiling_on_sc=False,
                         needs_layout_passes=False))


def _rsplat(root, bias, ipp, c):
    rs = jnp.broadcast_to(root.T[..., None], (c, ipp, 16))
    bi = jnp.broadcast_to(bias[:, None], (c, 16))
    return rs, bi


_EC = 12800


def _zmlp3_kernel(ea_ref, Wt_ref, b_ref, z1_ref, z2_ref, z3_ref):
    z = jnp.dot(Wt_ref[...], ea_ref[...],
                preferred_element_type=jnp.float32) + b_ref[...]
    z = jnp.maximum(z, 0.0)
    z1_ref[...] = z[:32]
    z2_ref[...] = z[32:96]
    z3_ref[...] = z[96:]


def _zmlp3(ea8, nn1_W, nn1_b, nn2_W, nn2_b, nn3_W, nn3_b):
    def prep(W, b, in_ch, out_ch):
        K = out_ch * in_ch
        Wp = W.reshape(F, in_ch, out_ch).transpose(2, 1, 0)
        return Wp.reshape(K, F), b.reshape(in_ch, out_ch).T.reshape(K, 1)

    W1p, b1p = prep(nn1_W, nn1_b, F, CS)
    W2p, b2p = prep(nn2_W, nn2_b, CS, CS)
    W3p, b3p = prep(nn3_W, nn3_b, CS, F)
    Wt = jnp.concatenate([W1p, W2p, W3p], axis=0)
    Wt = jnp.zeros((128, 8), jnp.float32).at[:, :F].set(Wt)
    bt = jnp.concatenate([b1p, b2p, b3p], axis=0)
    grid = (E // _EC,)
    return pl.pallas_call(
        _zmlp3_kernel,
        grid=grid,
        in_specs=[
            pl.BlockSpec((8, _EC), lambda i: (0, i)),
            pl.BlockSpec((128, 8), lambda i: (0, 0)),
            pl.BlockSpec((128, 1), lambda i: (0, 0)),
        ],
        out_specs=[
            pl.BlockSpec((32, _EC), lambda i: (0, i)),
            pl.BlockSpec((64, _EC), lambda i: (0, i)),
            pl.BlockSpec((32, _EC), lambda i: (0, i)),
        ],
        out_shape=[
            jax.ShapeDtypeStruct((32, E), jnp.float32),
            jax.ShapeDtypeStruct((64, E), jnp.float32),
            jax.ShapeDtypeStruct((32, E), jnp.float32),
        ],
    )(ea8, Wt, bt)


def _tail_kernel(p3_ref, cntp_ref, h2T_ref, root3t_ref, bias3_ref,
                 ts_ref, noisyT_ref,
                 tl1_W_ref, tl1_b_ref, tl2_W_ref, tl2_b_ref,
                 fm1_Wt_ref, fm1_b_ref, fm2_Wt_ref, fm2_b_ref,
                 fm3_Wt_ref, fm3_b_ref,
                 bn_gamma_ref, bn_beta_ref, out_ref):
    cnt = (cntp_ref[0:1, :NN] + cntp_ref[1:2, :NN])
    rc = 1.0 / jnp.maximum(cnt, 1.0)
    h2T = h2T_ref[...][:, :NN]
    mean = (p3_ref[0, :, :NN] + p3_ref[1, :, :NN]) * rc
    h3T = mean + jnp.dot(root3t_ref[...], h2T,
                         preferred_element_type=jnp.float32) + bias3_ref[...]

    t = ts_ref[...].astype(jnp.float32)
    half = N // 2
    k = lax.broadcasted_iota(jnp.int32, (1, half), 1).astype(jnp.float32)
    freqs = jnp.exp(k * (-(math.log(10000.0) / (half - 1))))
    ang = t * freqs
    se = jnp.concatenate([jnp.sin(ang), jnp.cos(ang)], axis=-1)
    pre = jnp.dot(se, tl1_W_ref[...], preferred_element_type=jnp.float32) + tl1_b_ref[...]
    g = 0.5 * pre * (1.0 + lax.erf(pre / jnp.sqrt(2.0).astype(jnp.float32)))
    te = jnp.dot(g, tl2_W_ref[...], preferred_element_type=jnp.float32) + tl2_b_ref[...]

    m1 = jax.nn.relu(jnp.dot(fm1_Wt_ref[...], h3T, preferred_element_type=jnp.float32)
                     + fm1_b_ref[...])
    m2 = jax.nn.relu(jnp.dot(fm2_Wt_ref[...], m1, preferred_element_type=jnp.float32)
                     + fm2_b_ref[...])
    m3 = jax.nn.sigmoid(jnp.dot(fm3_Wt_ref[...], m2, preferred_element_type=jnp.float32)
                        + fm3_b_ref[...])
    mapped = m3.reshape(F, B, N) * te[None, :, :]

    noisyT = noisyT_ref[...]
    mu = jnp.sum(noisyT, axis=(0, 1), keepdims=True) / (F * B)
    var = jnp.sum((noisyT - mu) ** 2, axis=(0, 1), keepdims=True) / (F * B)
    bn = (noisyT - mu) * lax.rsqrt(var + 1e-5)
    bn = bn * bn_gamma_ref[...][None, None, :] + bn_beta_ref[...][None, None, :]
    out_ref[...] = bn - mapped


def kernel(noisy_x, source_x, edge_index, source_edge_attr, timesteps,
           nn1_W, nn1_b, nn2_W, nn2_b, nn3_W, nn3_b,
           root1, bias1, root2, bias2, root3, bias3,
           tl1_W, tl1_b, tl2_W, tl2_b,
           fm1_W, fm1_b, fm2_W, fm2_b, fm3_W, fm3_b,
           bn_gamma, bn_beta):
    f32 = jnp.float32
    x = source_x.reshape(NN, F)
    xTp = jnp.zeros((F, NP), f32).at[:, :NN].set(x.T)
    src = edge_index[0]
    dst = edge_index[1]

    ea8 = jnp.zeros((8, E), f32).at[:F, :].set(
        source_edge_attr.reshape(E, F).T)
    zT1, zT2, zT3 = _zmlp3(ea8, nn1_W, nn1_b, nn2_W, nn2_b, nn3_W, nn3_b)
    rs1, bi1 = _rsplat(root1, bias1, F, CS)
    rs2, bi2 = _rsplat(root2, bias2, CS, CS)

    p1, cntp = _sc_layer(F, CS, 0, True, False)(
        xTp, zT1, src, dst)
    p2, h1T = _sc_layer(CS, CS, F, False, True)(
        p1, cntp, xTp, rs1, bi1, zT2, src, dst)
    p3, h2T = _sc_layer(CS, F, CS, False, True)(
        p2, cntp, h1T, rs2, bi2, zT3, src, dst)

    noisyT = noisy_x.transpose(2, 0, 1)
    outT = pl.pallas_call(
        _tail_kernel,
        out_shape=jax.ShapeDtypeStruct((F, B, N), f32),
    )(p3, cntp, h2T, root3.T, bias3.reshape(F, 1),
      timesteps.reshape(B, 1), noisyT,
      tl1_W, tl1_b.reshape(1, N), tl2_W, tl2_b.reshape(1, N),
      fm1_W.T, fm1_b.reshape(128, 1), fm2_W.T, fm2_b.reshape(128, 1),
      fm3_W.T, fm3_b.reshape(F, 1), bn_gamma, bn_beta)
    return outT.transpose(1, 2, 0)


# speedup vs baseline: 1.3166x; 1.0174x over previous
import math
import functools
import jax
import jax.numpy as jnp
from jax import lax
from jax.experimental import pallas as pl
from jax.experimental.pallas import tpu as pltpu
from jax.experimental.pallas import tpu_sc as plsc

B = 2
N = 400
F = 4
CS = 8
EPG = N * N
E = B * EPG
NN = B * N

NP = 1024
NW = 32
EPT = E // NW
NG = NP // 16
NGV = NN // 16
COLS = NP // 16
RC = 16
def _seg_plan(in_ch):
    segs = [1600] * 6 + [400]
    offs = [sum(segs[:k]) for k in range(len(segs))]
    return segs, offs, 2


def _sc_layer_body(in_ch, out_ch, ipp, want_counts, want_h, *refs):
    finalize = ipp > 0
    it = iter(refs)
    if finalize:
        pprev_r = next(it); cntp_r = next(it); hroot_r = next(it)
        rs_r = next(it); bi_r = next(it)
    else:
        xT_r = next(it)
    zT_r = next(it); src_r = next(it); dst_r = next(it)
    p_out = next(it)
    cnt_out = next(it) if want_counts else None
    h_out = next(it) if want_h else None
    hbuf = next(it); srcb = next(it); dstb = next(it)
    wbuf = next(it); sems = next(it)
    acc2d = next(it)
    if finalize:
        pv = next(it); cv = next(it); hv = next(it)
        rsv = next(it); biv = next(it)
    if want_counts:
        cnt1d = next(it)
    SA = next(it)
    SA2 = next(it) if want_counts else None
    rbuf = next(it); rsum = next(it)
    if want_counts:
        rbufc = next(it); rsumc = next(it)

    cid = lax.axis_index("c")
    sid = lax.axis_index("s")
    wid = sid * 2 + cid
    e0 = wid * EPT

    pltpu.sync_copy(src_r.at[pl.ds(e0, EPT)], srcb)
    pltpu.sync_copy(dst_r.at[pl.ds(e0, EPT)], dstb)
    if finalize:
        pltpu.sync_copy(pprev_r, pv)
        pltpu.sync_copy(cntp_r, cv)
        pltpu.sync_copy(hroot_r, hv)
        pltpu.sync_copy(rs_r, rsv)
        pltpu.sync_copy(bi_r, biv)
    else:
        pltpu.sync_copy(xT_r, hbuf)

    if finalize:
        rvecs = [[rsv[c, i, :] for i in range(ipp)] for c in range(in_ch)]
        bvecs = [biv[c, :] for c in range(in_ch)]

        @pl.loop(0, NG)
        def _fin(g):
            sl = pl.ds(g * 16, 16)
            cnt = cv[0, sl] + cv[1, sl]
            rc = 1.0 / jnp.maximum(cnt, 1.0)
            for c in range(in_ch):
                s = (pv[0, c, sl] + pv[1, c, sl]) * rc + bvecs[c]
                for i in range(ipp):
                    s = s + hv[i, sl] * rvecs[c][i]
                hbuf[c, sl] = s

    z16 = jnp.zeros((16,), jnp.float32)

    @pl.loop(0, NG)
    def _zero2d(g):
        sl = pl.ds(g * 16, 16)
        for o in range(out_ch):
            acc2d[o, sl] = z16
        if want_counts:
            cnt1d[sl] = z16

    lane = lax.iota(jnp.int32, 16)
    ones = jnp.full((16,), 1.0, jnp.float32)

    SEGS, SOFF, UNR = _seg_plan(in_ch)
    jidx = [jnp.full((16,), j, jnp.int32) for j in range(out_ch)]
    for p in range(out_ch // 2):
        o0 = 2 * p
        fuse_counts = want_counts and p == 0

        copies = [None] * len(SEGS)

        def _start(s):
            return pltpu.async_copy(
                zT_r.at[pl.ds(o0 * in_ch, 2 * in_ch),
                        pl.ds(e0 + SOFF[s], SEGS[s])],
                wbuf.at[s % 2, :, pl.ds(0, SEGS[s])], sems[s % 2])

        copies[0] = _start(0)
        for s in range(len(SEGS)):
            copies[s].wait()
            if s + 1 < len(SEGS):
                copies[s + 1] = _start(s + 1)
            bsl = s % 2

            @pl.loop(0, SEGS[s] // 16, unroll=UNR)
            def _edges(t):
                sle = pl.ds(SOFF[s] + t * 16, 16)
                slw = pl.ds(t * 16, 16)
                sv = srcb[sle]
                dv = dstb[sle]
                msg0 = z16
                msg1 = z16
                for i in range(in_ch):
                    xj = plsc.load_gather(
                        hbuf, [jnp.full((16,), i, jnp.int32), sv])
                    msg0 = msg0 + xj * wbuf[bsl, i, slw]
                    msg1 = msg1 + xj * wbuf[bsl, in_ch + i, slw]
                plsc.addupdate_scatter(acc2d, [jidx[o0], dv], msg0)
                plsc.addupdate_scatter(acc2d, [jidx[o0 + 1], dv], msg1)
                if fuse_counts:
                    plsc.addupdate_scatter(cnt1d, [dv], ones)

    pltpu.sync_copy(acc2d, SA.at[sid])
    if want_counts:
        pltpu.sync_copy(cnt1d, SA2.at[sid])
    plsc.subcore_barrier()

    for q in range(COLS // RC):
        c0 = sid * COLS + q * RC
        pltpu.sync_copy(SA.at[:, :, pl.ds(c0, RC)], rbuf)
        for o in range(out_ch):
            s = rbuf[0, o, :]
            for t in range(1, 16):
                s = s + rbuf[t, o, :]
            rsum[o, :] = s
        pltpu.sync_copy(rsum, p_out.at[cid, :, pl.ds(c0, RC)])
        if want_counts:
            pltpu.sync_copy(SA2.at[:, pl.ds(c0, RC)], rbufc)
            s = rbufc[0, :]
            for t in range(1, 16):
                s = s + rbufc[t, :]
            rsumc[:] = s
            pltpu.sync_copy(rsumc, cnt_out.at[cid, pl.ds(c0, RC)])

    if want_h:
        @pl.when(jnp.logical_and(cid == 0, sid == 0))
        def _wh():
            pltpu.sync_copy(hbuf, h_out)


def _sc_layer(in_ch, out_ch, ipp, want_counts, want_h):
    finalize = ipp > 0
    f32 = jnp.float32
    out_type = [jax.ShapeDtypeStruct((2, out_ch, NP), f32)]
    if want_counts:
        out_type.append(jax.ShapeDtypeStruct((2, NP), f32))
    if want_h:
        out_type.append(jax.ShapeDtypeStruct((in_ch, NP), f32))
    scratch = [
        pltpu.VMEM((in_ch, NP), f32),
        pltpu.VMEM((EPT,), jnp.int32),
        pltpu.VMEM((EPT,), jnp.int32),
        pltpu.VMEM((2, 2 * in_ch, _seg_plan(in_ch)[0][0]), f32),
        (pltpu.SemaphoreType.DMA, pltpu.SemaphoreType.DMA),
        pltpu.VMEM((out_ch, NP), f32),
    ]
    if finalize:
        scratch += [
            pltpu.VMEM((2, in_ch, NP), f32),
            pltpu.VMEM((2, NP), f32),
            pltpu.VMEM((ipp, NP), f32),
            pltpu.VMEM((in_ch, ipp, 16), f32),
            pltpu.VMEM((in_ch, 16), f32),
        ]
    if want_counts:
        scratch.append(pltpu.VMEM((NP,), f32))
    scratch.append(pltpu.VMEM_SHARED((16, out_ch, NP), f32))
    if want_counts:
        scratch.append(pltpu.VMEM_SHARED((16, NP), f32))
    scratch += [
        pltpu.VMEM((16, out_ch, RC), f32),
        pltpu.VMEM((out_ch, RC), f32),
    ]
    if want_counts:
        scratch += [
            pltpu.VMEM((16, RC), f32),
            pltpu.VMEM((RC,), f32),
        ]
    mesh = plsc.VectorSubcoreMesh(core_axis_name="c", subcore_axis_name="s")
    body = functools.partial(_sc_layer_body, in_ch, out_ch, ipp,
                             want_counts, want_h)
    return pl.kernel(body, out_type=out_type, mesh=mesh,
                     scratch_types=scratch,
                     compiler_params=pltpu.CompilerParams(
                         use_tc_tiling_on_sc=False,
                         needs_layout_passes=False))


def _rsplat(root, bias, ipp, c):
    rs = jnp.broadcast_to(root.T[..., None], (c, ipp, 16))
    bi = jnp.broadcast_to(bias[:, None], (c, 16))
    return rs, bi


_EC = 12800


def _zmlp3_kernel(ea_ref, Wt_ref, b_ref, z1_ref, z2_ref, z3_ref):
    z = jnp.dot(Wt_ref[...], ea_ref[...],
                preferred_element_type=jnp.float32) + b_ref[...]
    z = jnp.maximum(z, 0.0)
    z1_ref[...] = z[:32]
    z2_ref[...] = z[32:96]
    z3_ref[...] = z[96:]


def _zmlp3(ea8, nn1_W, nn1_b, nn2_W, nn2_b, nn3_W, nn3_b):
    def prep(W, b, in_ch, out_ch):
        K = out_ch * in_ch
        Wp = W.reshape(F, in_ch, out_ch).transpose(2, 1, 0)
        return Wp.reshape(K, F), b.reshape(in_ch, out_ch).T.reshape(K, 1)

    W1p, b1p = prep(nn1_W, nn1_b, F, CS)
    W2p, b2p = prep(nn2_W, nn2_b, CS, CS)
    W3p, b3p = prep(nn3_W, nn3_b, CS, F)
    Wt = jnp.concatenate([W1p, W2p, W3p], axis=0)
    Wt = jnp.zeros((128, 8), jnp.float32).at[:, :F].set(Wt)
    bt = jnp.concatenate([b1p, b2p, b3p], axis=0)
    grid = (E // _EC,)
    return pl.pallas_call(
        _zmlp3_kernel,
        grid=grid,
        in_specs=[
            pl.BlockSpec((8, _EC), lambda i: (0, i)),
            pl.BlockSpec((128, 8), lambda i: (0, 0)),
            pl.BlockSpec((128, 1), lambda i: (0, 0)),
        ],
        out_specs=[
            pl.BlockSpec((32, _EC), lambda i: (0, i)),
            pl.BlockSpec((64, _EC), lambda i: (0, i)),
            pl.BlockSpec((32, _EC), lambda i: (0, i)),
        ],
        out_shape=[
            jax.ShapeDtypeStruct((32, E), jnp.float32),
            jax.ShapeDtypeStruct((64, E), jnp.float32),
            jax.ShapeDtypeStruct((32, E), jnp.float32),
        ],
    )(ea8, Wt, bt)


def _tail_kernel(p3_ref, cntp_ref, h2T_ref, root3t_ref, bias3_ref,
                 ts_ref, noisyT_ref,
                 tl1_W_ref, tl1_b_ref, tl2_W_ref, tl2_b_ref,
                 fm1_Wt_ref, fm1_b_ref, fm2_Wt_ref, fm2_b_ref,
                 fm3_Wt_ref, fm3_b_ref,
                 bn_gamma_ref, bn_beta_ref, out_ref):
    cnt = (cntp_ref[0:1, :NN] + cntp_ref[1:2, :NN])
    rc = 1.0 / jnp.maximum(cnt, 1.0)
    h2T = h2T_ref[...][:, :NN]
    mean = (p3_ref[0, :, :NN] + p3_ref[1, :, :NN]) * rc
    h3T = mean + jnp.dot(root3t_ref[...], h2T,
                         preferred_element_type=jnp.float32) + bias3_ref[...]

    t = ts_ref[...].astype(jnp.float32)
    half = N // 2
    k = lax.broadcasted_iota(jnp.int32, (1, half), 1).astype(jnp.float32)
    freqs = jnp.exp(k * (-(math.log(10000.0) / (half - 1))))
    ang = t * freqs
    se = jnp.concatenate([jnp.sin(ang), jnp.cos(ang)], axis=-1)
    pre = jnp.dot(se, tl1_W_ref[...], preferred_element_type=jnp.float32) + tl1_b_ref[...]
    g = 0.5 * pre * (1.0 + lax.erf(pre / jnp.sqrt(2.0).astype(jnp.float32)))
    te = jnp.dot(g, tl2_W_ref[...], preferred_element_type=jnp.float32) + tl2_b_ref[...]

    m1 = jax.nn.relu(jnp.dot(fm1_Wt_ref[...], h3T, preferred_element_type=jnp.float32)
                     + fm1_b_ref[...])
    m2 = jax.nn.relu(jnp.dot(fm2_Wt_ref[...], m1, preferred_element_type=jnp.float32)
                     + fm2_b_ref[...])
    m3 = jax.nn.sigmoid(jnp.dot(fm3_Wt_ref[...], m2, preferred_element_type=jnp.float32)
                        + fm3_b_ref[...])
    mapped = m3.reshape(F, B, N) * te[None, :, :]

    noisyT = noisyT_ref[...]
    mu = jnp.sum(noisyT, axis=(0, 1), keepdims=True) / (F * B)
    var = jnp.sum((noisyT - mu) ** 2, axis=(0, 1), keepdims=True) / (F * B)
    bn = (noisyT - mu) * lax.rsqrt(var + 1e-5)
    bn = bn * bn_gamma_ref[...][None, None, :] + bn_beta_ref[...][None, None, :]
    out_ref[...] = bn - mapped


def kernel(noisy_x, source_x, edge_index, source_edge_attr, timesteps,
           nn1_W, nn1_b, nn2_W, nn2_b, nn3_W, nn3_b,
           root1, bias1, root2, bias2, root3, bias3,
           tl1_W, tl1_b, tl2_W, tl2_b,
           fm1_W, fm1_b, fm2_W, fm2_b, fm3_W, fm3_b,
           bn_gamma, bn_beta):
    f32 = jnp.float32
    x = source_x.reshape(NN, F)
    xTp = jnp.zeros((F, NP), f32).at[:, :NN].set(x.T)
    src = edge_index[0]
    dst = edge_index[1]

    ea8 = jnp.zeros((8, E), f32).at[:F, :].set(
        source_edge_attr.reshape(E, F).T)
    zT1, zT2, zT3 = _zmlp3(ea8, nn1_W, nn1_b, nn2_W, nn2_b, nn3_W, nn3_b)
    rs1, bi1 = _rsplat(root1, bias1, F, CS)
    rs2, bi2 = _rsplat(root2, bias2, CS, CS)

    p1, cntp = _sc_layer(F, CS, 0, True, False)(
        xTp, zT1, src, dst)
    p2, h1T = _sc_layer(CS, CS, F, False, True)(
        p1, cntp, xTp, rs1, bi1, zT2, src, dst)
    p3, h2T = _sc_layer(CS, F, CS, False, True)(
        p2, cntp, h1T, rs2, bi2, zT3, src, dst)

    noisyT = noisy_x.transpose(2, 0, 1)
    outT = pl.pallas_call(
        _tail_kernel,
        out_shape=jax.ShapeDtypeStruct((F, B, N), f32),
    )(p3, cntp, h2T, root3.T, bias3.reshape(F, 1),
      timesteps.reshape(B, 1), noisyT,
      tl1_W, tl1_b.reshape(1, N), tl2_W, tl2_b.reshape(1, N),
      fm1_W.T, fm1_b.reshape(128, 1), fm2_W.T, fm2_b.reshape(128, 1),
      fm3_W.T, fm3_b.reshape(F, 1), bn_gamma, bn_beta)
    return outT.transpose(1, 2, 0)
